# baseline (speedup 1.0000x reference)
import functools

import jax
import jax.numpy as jnp
from jax import lax
from jax.experimental import pallas as pl
from jax.experimental.pallas import tpu as pltpu
from jax.experimental.pallas import tpu_sc as plsc

N_NODES = 10000
N_EDGES = 320000
D = 128

NC = 2
NS = 16
NW = NC * NS
EPW = N_EDGES // NW
CHUNK = 80
NCHUNK = EPW // CHUNK
ACHUNK = 96
AEPW = 10080
ANCHUNK = AEPW // ACHUNK
DEAD = N_NODES
A_ROWS = 10112
ARPT = A_ROWS // NS

_MESH = plsc.VectorSubcoreMesh(
    core_axis_name="c", subcore_axis_name="s", num_cores=NC)


@functools.partial(
    pl.kernel,
    out_type=jax.ShapeDtypeStruct((NC, A_ROWS, D), jnp.float32),
    mesh=_MESH,
    scratch_types=[
        pltpu.VMEM((AEPW,), jnp.int32),
        pltpu.VMEM((ANCHUNK, ACHUNK), jnp.int32),
        pltpu.VMEM((2, ACHUNK, D), jnp.float32),
        pltpu.VMEM_SHARED((A_ROWS, D), jnp.float32),
        pltpu.SemaphoreType.DMA,
        pltpu.SemaphoreType.DMA,
    ],
)
def _agg(x_hbm, src_hbm, dst_hbm, zeros_hbm, out_hbm, sidx_v, didx_v, rows_v,
         acc_sh, gsem0, gsem1):
  cid = lax.axis_index("c")
  sid = lax.axis_index("s")
  wid = sid * NC + cid

  pltpu.sync_copy(zeros_hbm, acc_sh.at[pl.ds(sid * ARPT, ARPT)])
  pltpu.sync_copy(src_hbm.at[wid], sidx_v)
  pltpu.sync_copy(dst_hbm.at[wid], didx_v)
  plsc.subcore_barrier()

  gsems = (gsem0, gsem1)

  def fire(j, buf):
    pltpu.async_copy(
        x_hbm.at[sidx_v.at[pl.ds(j * ACHUNK, ACHUNK)]], rows_v.at[buf],
        gsems[buf])

  fire(0, 0)
  fire(1, 1)

  def step(j, buf):
    pltpu.make_async_copy(
        x_hbm.at[sidx_v.at[pl.ds(j * ACHUNK, ACHUNK)]], rows_v.at[buf],
        gsems[buf]).wait()
    pltpu.sync_copy(rows_v.at[buf], acc_sh.at[didx_v.at[j]], add=True)

    @pl.when(j + 2 < ANCHUNK)
    def _():
      fire(j + 2, buf)

  def loop_body(jj, _):
    step(2 * jj, 0)
    step(2 * jj + 1, 1)
    return 0

  lax.fori_loop(0, ANCHUNK // 2, loop_body, 0)
  if ANCHUNK % 2:
    step(ANCHUNK - 1, 0)

  plsc.subcore_barrier()
  pltpu.sync_copy(
      acc_sh.at[pl.ds(sid * ARPT, ARPT)],
      out_hbm.at[cid, pl.ds(sid * ARPT, ARPT)])


@functools.partial(
    pl.kernel,
    out_type=jax.ShapeDtypeStruct((NC, A_ROWS, D), jnp.float32),
    mesh=_MESH,
    scratch_types=[
        pltpu.VMEM((NCHUNK, CHUNK), jnp.int32),
        pltpu.VMEM((CHUNK, D), jnp.float32),
        pltpu.VMEM_SHARED((A_ROWS, D), jnp.float32),
        pltpu.SemaphoreType.DMA,
    ],
)
def _deg(ei_hbm, ones_hbm, zeros_hbm, out_hbm, idx_v, ones_v, acc_sh, sem):
  cid = lax.axis_index("c")
  sid = lax.axis_index("s")
  wid = sid * NC + cid

  pltpu.sync_copy(zeros_hbm, acc_sh.at[pl.ds(sid * ARPT, ARPT)])
  pltpu.sync_copy(ei_hbm.at[1, wid], idx_v)
  pltpu.sync_copy(ones_hbm, ones_v)
  plsc.subcore_barrier()

  G = 5

  def loop_body(g, _):
    for b in range(G):
      pltpu.async_copy(ones_v, acc_sh.at[idx_v.at[G * g + b]], sem, add=True)
    for b in range(G):
      pltpu.make_async_copy(ones_v, acc_sh.at[idx_v.at[G * g + b]], sem).wait()
    return 0

  lax.fori_loop(0, NCHUNK // G, loop_body, 0)

  plsc.subcore_barrier()
  pltpu.sync_copy(
      acc_sh.at[pl.ds(sid * ARPT, ARPT)],
      out_hbm.at[cid, pl.ds(sid * ARPT, ARPT)])


_BLK = 1000


def _tc_body(parts_ref, deg_ref, w_ref, b_ref, h_ref):
  s = parts_ref[0] + parts_ref[1]
  h = lax.dot_general(s, w_ref[...], (((1,), (1,)), ((), ())),
                      preferred_element_type=jnp.float32)
  deg = deg_ref[0, :, :1] + deg_ref[1, :, :1]
  h_ref[...] = h + deg * b_ref[...]


def _tc_layer(parts, deg, W, b):
  return pl.pallas_call(
      _tc_body,
      grid=(N_NODES // _BLK,),
      in_specs=[
          pl.BlockSpec((NC, _BLK, D), lambda i: (0, i, 0)),
          pl.BlockSpec((NC, _BLK, D), lambda i: (0, i, 0)),
          pl.BlockSpec((D, D), lambda i: (0, 0)),
          pl.BlockSpec((D,), lambda i: (0,)),
      ],
      out_specs=pl.BlockSpec((_BLK, D), lambda i: (i, 0)),
      out_shape=jax.ShapeDtypeStruct((N_NODES, D), jnp.float32),
  )(parts, deg, W, b)


def kernel(x, edge_index, W0, b0, W1, b1):
  ei = edge_index.astype(jnp.int32)
  ei80 = ei.reshape(2, NW, NCHUNK, CHUNK)
  eiw = ei.reshape(2, NW, EPW)
  pad = jnp.zeros((2, NW, AEPW - EPW), jnp.int32).at[1].set(DEAD)
  eip = jnp.concatenate([eiw, pad], axis=2)
  src2 = eip[0]
  dst3 = eip[1].reshape(NW, ANCHUNK, ACHUNK)
  z = jnp.zeros((ARPT, D), jnp.float32)
  ones = jnp.ones((CHUNK, D), jnp.float32)

  deg = _deg(ei80, ones, z)
  parts1 = _agg(x, src2, dst3, z)
  h1 = _tc_layer(parts1, deg, W0, b0)
  parts2 = _agg(h1, src2, dst3, z)
  h2 = _tc_layer(parts2, deg, W1, b1)
  return h2


# speedup vs baseline: 1.4830x; 1.4830x over previous
import functools

import jax
import jax.numpy as jnp
from jax import lax
from jax.experimental import pallas as pl
from jax.experimental.pallas import tpu as pltpu
from jax.experimental.pallas import tpu_sc as plsc

N_NODES = 10000
N_EDGES = 320000
D = 128

NC = 2
NS = 16
NW = NC * NS
EPW = N_EDGES // NW
CHUNK = 80
NCHUNK = EPW // CHUNK
A_ROWS = 10112
ARPT = A_ROWS // NS

_MESH = plsc.VectorSubcoreMesh(
    core_axis_name="c", subcore_axis_name="s", num_cores=NC)


@functools.partial(
    pl.kernel,
    out_type=jax.ShapeDtypeStruct((NC, A_ROWS, D), jnp.float32),
    mesh=_MESH,
    scratch_types=[
        pltpu.VMEM((EPW,), jnp.int32),
        pltpu.VMEM((NCHUNK, CHUNK), jnp.int32),
        pltpu.VMEM((2, CHUNK, D), jnp.float32),
        pltpu.VMEM_SHARED((A_ROWS, D), jnp.float32),
        pltpu.SemaphoreType.DMA,
        pltpu.SemaphoreType.DMA,
    ],
)
def _agg(x_hbm, src_hbm, dst_hbm, zeros_hbm, out_hbm, sidx_v, didx_v, rows_v,
         acc_sh, gsem0, gsem1):
  cid = lax.axis_index("c")
  sid = lax.axis_index("s")
  wid = sid * NC + cid

  pltpu.sync_copy(zeros_hbm, acc_sh.at[pl.ds(sid * ARPT, ARPT)])
  pltpu.sync_copy(src_hbm.at[wid], sidx_v)
  pltpu.sync_copy(dst_hbm.at[wid], didx_v)
  plsc.subcore_barrier()

  gsems = (gsem0, gsem1)

  def fire(j, buf):
    pltpu.async_copy(
        x_hbm.at[sidx_v.at[pl.ds(j * CHUNK, CHUNK)]], rows_v.at[buf],
        gsems[buf])

  fire(0, 0)
  fire(1, 1)

  def step(j, buf):
    pltpu.make_async_copy(
        x_hbm.at[sidx_v.at[pl.ds(j * CHUNK, CHUNK)]], rows_v.at[buf],
        gsems[buf]).wait()
    pltpu.sync_copy(rows_v.at[buf], acc_sh.at[didx_v.at[j]], add=True)

    @pl.when(j + 2 < NCHUNK)
    def _():
      fire(j + 2, buf)

  def loop_body(jj, _):
    step(2 * jj, 0)
    step(2 * jj + 1, 1)
    return 0

  lax.fori_loop(0, NCHUNK // 2, loop_body, 0)
  if NCHUNK % 2:
    step(NCHUNK - 1, 0)

  plsc.subcore_barrier()
  pltpu.sync_copy(
      acc_sh.at[pl.ds(sid * ARPT, ARPT)],
      out_hbm.at[cid, pl.ds(sid * ARPT, ARPT)])


@functools.partial(
    pl.kernel,
    out_type=jax.ShapeDtypeStruct((NC, A_ROWS, D), jnp.float32),
    mesh=_MESH,
    scratch_types=[
        pltpu.VMEM((NCHUNK, CHUNK), jnp.int32),
        pltpu.VMEM((CHUNK, D), jnp.float32),
        pltpu.VMEM_SHARED((A_ROWS, D), jnp.float32),
        pltpu.SemaphoreType.DMA,
    ],
)
def _deg(ei_hbm, ones_hbm, zeros_hbm, out_hbm, idx_v, ones_v, acc_sh, sem):
  cid = lax.axis_index("c")
  sid = lax.axis_index("s")
  wid = sid * NC + cid

  pltpu.sync_copy(zeros_hbm, acc_sh.at[pl.ds(sid * ARPT, ARPT)])
  pltpu.sync_copy(ei_hbm.at[1, wid], idx_v)
  pltpu.sync_copy(ones_hbm, ones_v)
  plsc.subcore_barrier()

  G = 5

  def loop_body(g, _):
    for b in range(G):
      pltpu.async_copy(ones_v, acc_sh.at[idx_v.at[G * g + b]], sem, add=True)
    for b in range(G):
      pltpu.make_async_copy(ones_v, acc_sh.at[idx_v.at[G * g + b]], sem).wait()
    return 0

  lax.fori_loop(0, NCHUNK // G, loop_body, 0)

  plsc.subcore_barrier()
  pltpu.sync_copy(
      acc_sh.at[pl.ds(sid * ARPT, ARPT)],
      out_hbm.at[cid, pl.ds(sid * ARPT, ARPT)])


_BLK = 1000


def _tc_body(parts_ref, deg_ref, w_ref, b_ref, h_ref):
  s = parts_ref[0] + parts_ref[1]
  h = lax.dot_general(s, w_ref[...], (((1,), (1,)), ((), ())),
                      preferred_element_type=jnp.float32)
  deg = deg_ref[0, :, :1] + deg_ref[1, :, :1]
  h_ref[...] = h + deg * b_ref[...]


def _tc_layer(parts, deg, W, b):
  return pl.pallas_call(
      _tc_body,
      grid=(N_NODES // _BLK,),
      in_specs=[
          pl.BlockSpec((NC, _BLK, D), lambda i: (0, i, 0)),
          pl.BlockSpec((NC, _BLK, D), lambda i: (0, i, 0)),
          pl.BlockSpec((D, D), lambda i: (0, 0)),
          pl.BlockSpec((D,), lambda i: (0,)),
      ],
      out_specs=pl.BlockSpec((_BLK, D), lambda i: (i, 0)),
      out_shape=jax.ShapeDtypeStruct((N_NODES, D), jnp.float32),
  )(parts, deg, W, b)


def kernel(x, edge_index, W0, b0, W1, b1):
  ei = edge_index.astype(jnp.int32)
  ei80 = ei.reshape(2, NW, NCHUNK, CHUNK)
  src2 = ei[0].reshape(NW, EPW)
  dst3 = ei[1].reshape(NW, NCHUNK, CHUNK)
  z = jnp.zeros((ARPT, D), jnp.float32)
  ones = jnp.ones((CHUNK, D), jnp.float32)

  deg = _deg(ei80, ones, z)
  parts1 = _agg(x, src2, dst3, z)
  h1 = _tc_layer(parts1, deg, W0, b0)
  parts2 = _agg(h1, src2, dst3, z)
  h2 = _tc_layer(parts2, deg, W1, b1)
  return h2
